# baseline (speedup 1.0000x reference)
import functools

import jax
import jax.numpy as jnp
from jax import lax
from jax.experimental import pallas as pl
from jax.experimental.pallas import tpu as pltpu
from jax.experimental.pallas import tpu_sc as plsc

N = 10000
E = 320000
NFEAT = 128
NHID = 128
NCLASS = 64

NC = 2
NS = 16
CHUNK = 128
CHUNKS_PER_TILE = 160
E_PAD = NS * CHUNK * CHUNKS_PER_TILE
HALF_CHUNKS = CHUNKS_PER_TILE // 2
NPAD = 10240
ROWS_PER_TILE = NPAD // NS
ZROWS = 64


def _sc_aggregate(cw, with_degree):
    mesh = plsc.VectorSubcoreMesh(core_axis_name="c", subcore_axis_name="s")
    out_type = [jax.ShapeDtypeStruct((NC, NPAD, cw), jnp.float32)]
    scratch = [
        pltpu.VMEM_SHARED((NPAD, cw), jnp.float32),
        pltpu.VMEM((CHUNKS_PER_TILE, CHUNK), jnp.int32),
        pltpu.VMEM((CHUNKS_PER_TILE, CHUNK), jnp.int32),
        pltpu.VMEM((CHUNK, cw), jnp.float32),
        pltpu.VMEM((CHUNK, cw), jnp.float32),
        pltpu.VMEM((ZROWS, cw), jnp.float32),
        pltpu.SemaphoreType.DMA,
        pltpu.SemaphoreType.DMA,
        pltpu.SemaphoreType.DMA,
    ]
    if with_degree:
        out_type.append(jax.ShapeDtypeStruct((NC, NPAD, 16), jnp.float32))
        scratch += [
            pltpu.VMEM_SHARED((NPAD, 16), jnp.float32),
            pltpu.VMEM((CHUNK, 16), jnp.float32),
            pltpu.VMEM((ZROWS, 16), jnp.float32),
        ]

    @functools.partial(
        pl.kernel,
        out_type=tuple(out_type),
        mesh=mesh,
        scratch_types=tuple(scratch),
        compiler_params=pltpu.CompilerParams(use_tc_tiling_on_sc=False),
    )
    def k(table_hbm, src4_hbm, dst3_hbm, *refs):
        if with_degree:
            (out_hbm, deg_hbm, acc_sh, src_v, dst_v, r0, r1, zbuf,
             g0, g1, zsem, deg_sh, ones_v, zbufd) = refs
        else:
            (out_hbm, acc_sh, src_v, dst_v, r0, r1, zbuf,
             g0, g1, zsem) = refs
        rows = [r0, r1]
        gsem = [g0, g1]

        cid = lax.axis_index("c")
        sid = lax.axis_index("s")

        z = jnp.zeros((16,), jnp.float32)

        def fill_z(i, _):
            for j in range(cw // 16):
                zbuf[i, pl.ds(j * 16, 16)] = z
            if with_degree:
                zbufd[i, :] = z
            return 0

        lax.fori_loop(0, ZROWS, fill_z, 0)

        if with_degree:
            one = jnp.ones((16,), jnp.float32)

            def fill_ones(i, _):
                ones_v[i, :] = one
                return 0

            lax.fori_loop(0, CHUNK, fill_ones, 0)

        row0 = sid * ROWS_PER_TILE
        nz = ROWS_PER_TILE // ZROWS

        def zero_body(i, _):
            pltpu.async_copy(zbuf, acc_sh.at[pl.ds(row0 + i * ZROWS, ZROWS)],
                             zsem)
            if with_degree:
                pltpu.async_copy(
                    zbufd, deg_sh.at[pl.ds(row0 + i * ZROWS, ZROWS)], zsem)
            return 0

        lax.fori_loop(0, nz, zero_body, 0)

        pltpu.sync_copy(src4_hbm.at[cid, sid], src_v)
        pltpu.sync_copy(dst3_hbm.at[sid], dst_v)

        def zero_drain(i, _):
            pltpu.make_async_copy(
                zbuf, acc_sh.at[pl.ds(row0, ZROWS)], zsem).wait()
            if with_degree:
                pltpu.make_async_copy(
                    zbufd, deg_sh.at[pl.ds(row0, ZROWS)], zsem).wait()
            return 0

        lax.fori_loop(0, nz, zero_drain, 0)

        plsc.subcore_barrier()

        def fire_g(c, k):
            pltpu.async_copy(table_hbm.at[src_v.at[c]], rows[k], gsem[k])

        def wait_g(k):
            pltpu.make_async_copy(
                table_hbm.at[src_v.at[0]], rows[k], gsem[k]).wait()

        def scat(c, k):
            pltpu.sync_copy(rows[k], acc_sh.at[dst_v.at[c]], add=True)
            if with_degree:
                do = jnp.logical_or(
                    jnp.logical_and(cid == 0, c < HALF_CHUNKS),
                    jnp.logical_and(cid != 0, c >= HALF_CHUNKS))

                @pl.when(do)
                def _():
                    pltpu.sync_copy(ones_v, deg_sh.at[dst_v.at[c]], add=True)

        fire_g(0, 0)
        NPAIR = CHUNKS_PER_TILE // 2

        def pair_body(g, _):
            c0 = 2 * g
            fire_g(c0 + 1, 1)
            wait_g(0)
            scat(c0, 0)

            @pl.when(g + 1 < NPAIR)
            def _():
                fire_g(c0 + 2, 0)

            wait_g(1)
            scat(c0 + 1, 1)
            return 0

        lax.fori_loop(0, NPAIR, pair_body, 0)

        plsc.subcore_barrier()

        pltpu.async_copy(
            acc_sh.at[pl.ds(row0, ROWS_PER_TILE)],
            out_hbm.at[cid, pl.ds(row0, ROWS_PER_TILE)],
            zsem)
        if with_degree:
            pltpu.async_copy(
                deg_sh.at[pl.ds(row0, ROWS_PER_TILE)],
                deg_hbm.at[cid, pl.ds(row0, ROWS_PER_TILE)],
                zsem)
            pltpu.make_async_copy(
                deg_sh.at[pl.ds(row0, ROWS_PER_TILE)],
                deg_hbm.at[cid, pl.ds(row0, ROWS_PER_TILE)],
                zsem).wait()
        pltpu.make_async_copy(
            acc_sh.at[pl.ds(row0, ROWS_PER_TILE)],
            out_hbm.at[cid, pl.ds(row0, ROWS_PER_TILE)],
            zsem).wait()

    return k


_sc_agg_l1 = _sc_aggregate(NFEAT // NC, with_degree=True)
_sc_agg_l2 = _sc_aggregate(NCLASS // NC, with_degree=False)

BR = 512
CW1 = NFEAT // NC
CW2 = NCLASS // NC


def _tc_layer1_body(agg_ref, deg_ref, x_ref, w1l_ref, b1_ref, w1r_ref,
                    w2l_ref, w2r_ref, hl_ref, hr_ref):
    agg = jnp.concatenate([agg_ref[0], agg_ref[1]], axis=1)
    deg = deg_ref[0, :, :1] + deg_ref[1, :, :1]
    inv = 1.0 / jnp.maximum(deg, 1.0)
    mean = agg * inv
    dn = (((1,), (1,)), ((), ()))
    h = (lax.dot_general(mean, w1l_ref[...], dn,
                         preferred_element_type=jnp.float32)
         + b1_ref[...]
         + lax.dot_general(x_ref[...], w1r_ref[...], dn,
                           preferred_element_type=jnp.float32))
    hl = lax.dot_general(h, w2l_ref[...], dn,
                         preferred_element_type=jnp.float32)
    hl_ref[0] = hl[:, :CW2]
    hl_ref[1] = hl[:, CW2:]
    hr_ref[...] = lax.dot_general(h, w2r_ref[...], dn,
                                  preferred_element_type=jnp.float32)


def _tc_layer2_body(agg_ref, deg_ref, hr_ref, b2_ref, out_ref):
    agg = jnp.concatenate([agg_ref[0], agg_ref[1]], axis=1)
    deg = deg_ref[0, :, :1] + deg_ref[1, :, :1]
    inv = 1.0 / jnp.maximum(deg, 1.0)
    z = agg * inv + b2_ref[...] + hr_ref[...]
    m = jnp.max(z, axis=1, keepdims=True)
    lse = m + jnp.log(jnp.sum(jnp.exp(z - m), axis=1, keepdims=True))
    out_ref[...] = z - lse


def _tc_layer1(agg1, deg, x_pad, W1l, b1, W1r, W2l, W2r):
    grid = (NPAD // BR,)
    return pl.pallas_call(
        _tc_layer1_body,
        grid=grid,
        in_specs=[
            pl.BlockSpec((NC, BR, CW1), lambda r: (0, r, 0)),
            pl.BlockSpec((NC, BR, 16), lambda r: (0, r, 0)),
            pl.BlockSpec((BR, NFEAT), lambda r: (r, 0)),
            pl.BlockSpec((NHID, NFEAT), lambda r: (0, 0)),
            pl.BlockSpec((1, NHID), lambda r: (0, 0)),
            pl.BlockSpec((NHID, NFEAT), lambda r: (0, 0)),
            pl.BlockSpec((NCLASS, NHID), lambda r: (0, 0)),
            pl.BlockSpec((NCLASS, NHID), lambda r: (0, 0)),
        ],
        out_specs=[
            pl.BlockSpec((NC, BR, CW2), lambda r: (0, r, 0)),
            pl.BlockSpec((BR, NCLASS), lambda r: (r, 0)),
        ],
        out_shape=[
            jax.ShapeDtypeStruct((NC, NPAD, CW2), jnp.float32),
            jax.ShapeDtypeStruct((NPAD, NCLASS), jnp.float32),
        ],
    )(agg1, deg, x_pad, W1l, b1, W1r, W2l, W2r)


def _tc_layer2(agg2, deg, hr, b2):
    grid = (NPAD // BR,)
    return pl.pallas_call(
        _tc_layer2_body,
        grid=grid,
        in_specs=[
            pl.BlockSpec((NC, BR, CW2), lambda r: (0, r, 0)),
            pl.BlockSpec((NC, BR, 16), lambda r: (0, r, 0)),
            pl.BlockSpec((BR, NCLASS), lambda r: (r, 0)),
            pl.BlockSpec((1, NCLASS), lambda r: (0, 0)),
        ],
        out_specs=pl.BlockSpec((BR, NCLASS), lambda r: (r, 0)),
        out_shape=jax.ShapeDtypeStruct((NPAD, NCLASS), jnp.float32),
    )(agg2, deg, hr, b2)


@jax.jit
def kernel(x, edge_index, W1l, b1, W1r, W2l, b2, W2r):
    src = edge_index[0]
    dst = edge_index[1]
    pad = E_PAD - E
    srcp = jnp.concatenate([src, jnp.zeros((pad,), jnp.int32)])
    dummy = N + jnp.arange(pad, dtype=jnp.int32) % (NPAD - N)
    dstp = jnp.concatenate([dst, dummy])
    src3 = srcp.reshape(NS, CHUNKS_PER_TILE, CHUNK)
    src4 = jnp.stack([src3, src3 + NPAD])
    dst3 = dstp.reshape(NS, CHUNKS_PER_TILE, CHUNK)

    x_pad = jnp.pad(x, ((0, NPAD - N), (0, 0)))
    xcat = jnp.concatenate([x_pad[:, :CW1], x_pad[:, CW1:]], axis=0)

    agg1, deg = _sc_agg_l1(xcat, src4, dst3)

    hl, hr = _tc_layer1(agg1, deg, x_pad, W1l, b1.reshape(1, NHID), W1r,
                        W2l, W2r)

    (agg2,) = _sc_agg_l2(hl.reshape(NC * NPAD, CW2), src4, dst3)

    out = _tc_layer2(agg2, deg, hr, b2.reshape(1, NCLASS))
    return out[:N]


# speedup vs baseline: 1.8122x; 1.8122x over previous
import functools

import jax
import jax.numpy as jnp
from jax import lax
from jax.experimental import pallas as pl
from jax.experimental.pallas import tpu as pltpu
from jax.experimental.pallas import tpu_sc as plsc

N = 10000
E = 320000
NFEAT = 128
NHID = 128
NCLASS = 64

NC = 2
NS = 16
CHUNK = 128
CHUNKS_PER_TILE = 160
E_PAD = NS * CHUNK * CHUNKS_PER_TILE
HALF_CHUNKS = CHUNKS_PER_TILE // 2
NPAD = 10240
ROWS_PER_TILE = NPAD // NS
ZROWS = 64


def _sc_aggregate(cw, with_degree):
    mesh = plsc.VectorSubcoreMesh(core_axis_name="c", subcore_axis_name="s")
    out_type = [jax.ShapeDtypeStruct((NC, NPAD, cw), jnp.float32)]
    scratch = [
        pltpu.VMEM_SHARED((NPAD, cw), jnp.float32),
        pltpu.VMEM((CHUNKS_PER_TILE, CHUNK), jnp.int32),
        pltpu.VMEM((CHUNKS_PER_TILE, CHUNK), jnp.int32),
        pltpu.VMEM((CHUNK, cw), jnp.float32),
        pltpu.VMEM((CHUNK, cw), jnp.float32),
        pltpu.VMEM((ZROWS, cw), jnp.float32),
        pltpu.SemaphoreType.DMA,
        pltpu.SemaphoreType.DMA,
        pltpu.SemaphoreType.DMA,
    ]
    if with_degree:
        out_type.append(jax.ShapeDtypeStruct((NC, NPAD, 16), jnp.float32))
        scratch += [
            pltpu.VMEM_SHARED((NPAD, 16), jnp.float32),
            pltpu.VMEM((CHUNK, 16), jnp.float32),
            pltpu.VMEM((ZROWS, 16), jnp.float32),
        ]

    @functools.partial(
        pl.kernel,
        out_type=tuple(out_type),
        mesh=mesh,
        scratch_types=tuple(scratch),
        compiler_params=pltpu.CompilerParams(use_tc_tiling_on_sc=False),
    )
    def k(table_hbm, src4_hbm, dst3_hbm, *refs):
        if with_degree:
            (out_hbm, deg_hbm, acc_sh, src_v, dst_v, r0, r1, zbuf,
             g0, g1, zsem, deg_sh, ones_v, zbufd) = refs
        else:
            (out_hbm, acc_sh, src_v, dst_v, r0, r1, zbuf,
             g0, g1, zsem) = refs
        rows = [r0, r1]
        gsem = [g0, g1]

        cid = lax.axis_index("c")
        sid = lax.axis_index("s")

        z = jnp.zeros((16,), jnp.float32)

        def fill_z(i, _):
            for j in range(cw // 16):
                zbuf[i, pl.ds(j * 16, 16)] = z
            if with_degree:
                zbufd[i, :] = z
            return 0

        lax.fori_loop(0, ZROWS, fill_z, 0)

        if with_degree:
            one = jnp.ones((16,), jnp.float32)

            def fill_ones(i, _):
                ones_v[i, :] = one
                return 0

            lax.fori_loop(0, CHUNK, fill_ones, 0)

        row0 = sid * ROWS_PER_TILE
        nz = ROWS_PER_TILE // ZROWS

        def zero_body(i, _):
            pltpu.async_copy(zbuf, acc_sh.at[pl.ds(row0 + i * ZROWS, ZROWS)],
                             zsem)
            if with_degree:
                pltpu.async_copy(
                    zbufd, deg_sh.at[pl.ds(row0 + i * ZROWS, ZROWS)], zsem)
            return 0

        lax.fori_loop(0, nz, zero_body, 0)

        pltpu.sync_copy(src4_hbm.at[cid, sid], src_v)
        pltpu.sync_copy(dst3_hbm.at[sid], dst_v)

        def zero_drain(i, _):
            pltpu.make_async_copy(
                zbuf, acc_sh.at[pl.ds(row0, ZROWS)], zsem).wait()
            if with_degree:
                pltpu.make_async_copy(
                    zbufd, deg_sh.at[pl.ds(row0, ZROWS)], zsem).wait()
            return 0

        lax.fori_loop(0, nz, zero_drain, 0)

        plsc.subcore_barrier()

        def fire_g(c, k):
            pltpu.async_copy(table_hbm.at[src_v.at[c]], rows[k], gsem[k])

        def wait_g(k):
            pltpu.make_async_copy(
                table_hbm.at[src_v.at[0]], rows[k], gsem[k]).wait()

        def scat(c, k):
            pltpu.sync_copy(rows[k], acc_sh.at[dst_v.at[c]], add=True)
            if with_degree:
                do = jnp.logical_or(
                    jnp.logical_and(cid == 0, c < HALF_CHUNKS),
                    jnp.logical_and(cid != 0, c >= HALF_CHUNKS))

                @pl.when(do)
                def _():
                    pltpu.sync_copy(ones_v, deg_sh.at[dst_v.at[c]], add=True)

        fire_g(0, 0)
        NPAIR = CHUNKS_PER_TILE // 2

        def pair_body(g, _):
            c0 = 2 * g
            fire_g(c0 + 1, 1)
            wait_g(0)
            scat(c0, 0)

            @pl.when(g + 1 < NPAIR)
            def _():
                fire_g(c0 + 2, 0)

            wait_g(1)
            scat(c0 + 1, 1)
            return 0

        lax.fori_loop(0, NPAIR, pair_body, 0)

        plsc.subcore_barrier()

        pltpu.async_copy(
            acc_sh.at[pl.ds(row0, ROWS_PER_TILE)],
            out_hbm.at[cid, pl.ds(row0, ROWS_PER_TILE)],
            zsem)
        if with_degree:
            pltpu.async_copy(
                deg_sh.at[pl.ds(row0, ROWS_PER_TILE)],
                deg_hbm.at[cid, pl.ds(row0, ROWS_PER_TILE)],
                zsem)
            pltpu.make_async_copy(
                deg_sh.at[pl.ds(row0, ROWS_PER_TILE)],
                deg_hbm.at[cid, pl.ds(row0, ROWS_PER_TILE)],
                zsem).wait()
        pltpu.make_async_copy(
            acc_sh.at[pl.ds(row0, ROWS_PER_TILE)],
            out_hbm.at[cid, pl.ds(row0, ROWS_PER_TILE)],
            zsem).wait()

    return k


_sc_agg_l1 = _sc_aggregate(NFEAT // NC, with_degree=True)
_sc_agg_l2 = _sc_aggregate(NCLASS // NC, with_degree=False)

BR = 512
CW1 = NFEAT // NC
CW2 = NCLASS // NC


def _tc_layer1_body(agg_ref, deg_ref, x_ref, w1l_ref, b1_ref, w1r_ref,
                    w2l_ref, w2r_ref, hl_ref, hr_ref):
    agg = jnp.concatenate([agg_ref[0], agg_ref[1]], axis=1)
    deg = deg_ref[0, :, :1] + deg_ref[1, :, :1]
    inv = 1.0 / jnp.maximum(deg, 1.0)
    mean = agg * inv
    dn = (((1,), (1,)), ((), ()))
    h = (lax.dot_general(mean, w1l_ref[...], dn,
                         preferred_element_type=jnp.float32)
         + b1_ref[...]
         + lax.dot_general(x_ref[...], w1r_ref[...], dn,
                           preferred_element_type=jnp.float32))
    hl = lax.dot_general(h, w2l_ref[...], dn,
                         preferred_element_type=jnp.float32)
    hl_ref[0] = hl[:, :CW2]
    hl_ref[1] = hl[:, CW2:]
    hr_ref[...] = lax.dot_general(h, w2r_ref[...], dn,
                                  preferred_element_type=jnp.float32)


def _tc_layer2_body(agg_ref, deg_ref, hr_ref, b2_ref, out_ref):
    agg = jnp.concatenate([agg_ref[0], agg_ref[1]], axis=1)
    deg = deg_ref[0, :, :1] + deg_ref[1, :, :1]
    inv = 1.0 / jnp.maximum(deg, 1.0)
    z = agg * inv + b2_ref[...] + hr_ref[...]
    m = jnp.max(z, axis=1, keepdims=True)
    lse = m + jnp.log(jnp.sum(jnp.exp(z - m), axis=1, keepdims=True))
    out_ref[...] = z - lse


def _tc_layer1(agg1, deg, x_pad, W1l, b1, W1r, W2l, W2r):
    grid = (NPAD // BR,)
    return pl.pallas_call(
        _tc_layer1_body,
        grid=grid,
        in_specs=[
            pl.BlockSpec((NC, BR, CW1), lambda r: (0, r, 0)),
            pl.BlockSpec((NC, BR, 16), lambda r: (0, r, 0)),
            pl.BlockSpec((BR, NFEAT), lambda r: (r, 0)),
            pl.BlockSpec((NHID, NFEAT), lambda r: (0, 0)),
            pl.BlockSpec((1, NHID), lambda r: (0, 0)),
            pl.BlockSpec((NHID, NFEAT), lambda r: (0, 0)),
            pl.BlockSpec((NCLASS, NHID), lambda r: (0, 0)),
            pl.BlockSpec((NCLASS, NHID), lambda r: (0, 0)),
        ],
        out_specs=[
            pl.BlockSpec((NC, BR, CW2), lambda r: (0, r, 0)),
            pl.BlockSpec((BR, NCLASS), lambda r: (r, 0)),
        ],
        out_shape=[
            jax.ShapeDtypeStruct((NC, NPAD, CW2), jnp.float32),
            jax.ShapeDtypeStruct((NPAD, NCLASS), jnp.float32),
        ],
    )(agg1, deg, x_pad, W1l, b1, W1r, W2l, W2r)


def _tc_layer2(agg2, deg, hr, b2):
    grid = (NPAD // BR,)
    return pl.pallas_call(
        _tc_layer2_body,
        grid=grid,
        in_specs=[
            pl.BlockSpec((NC, BR, CW2), lambda r: (0, r, 0)),
            pl.BlockSpec((NC, BR, 16), lambda r: (0, r, 0)),
            pl.BlockSpec((BR, NCLASS), lambda r: (r, 0)),
            pl.BlockSpec((1, NCLASS), lambda r: (0, 0)),
        ],
        out_specs=pl.BlockSpec((BR, NCLASS), lambda r: (r, 0)),
        out_shape=jax.ShapeDtypeStruct((NPAD, NCLASS), jnp.float32),
    )(agg2, deg, hr, b2)


@jax.jit
def kernel(x, edge_index, W1l, b1, W1r, W2l, b2, W2r):
    src = edge_index[0]
    dst = edge_index[1]
    pad = E_PAD - E
    srcp = jnp.concatenate([src, jnp.arange(pad, dtype=jnp.int32) % N])
    dummy = N + jnp.arange(pad, dtype=jnp.int32) % (NPAD - N)
    dstp = jnp.concatenate([dst, dummy])
    src3 = srcp.reshape(NS, CHUNKS_PER_TILE, CHUNK)
    src4 = jnp.stack([src3, src3 + NPAD])
    dst3 = dstp.reshape(NS, CHUNKS_PER_TILE, CHUNK)

    x_pad = jnp.pad(x, ((0, NPAD - N), (0, 0)))
    xcat = jnp.concatenate([x_pad[:, :CW1], x_pad[:, CW1:]], axis=0)

    agg1, deg = _sc_agg_l1(xcat, src4, dst3)

    hl, hr = _tc_layer1(agg1, deg, x_pad, W1l, b1.reshape(1, NHID), W1r,
                        W2l, W2r)

    (agg2,) = _sc_agg_l2(hl.reshape(NC * NPAD, CW2), src4, dst3)

    out = _tc_layer2(agg2, deg, hr, b2.reshape(1, NCLASS))
    return out[:N]


# speedup vs baseline: 1.8380x; 1.0142x over previous
import functools

import jax
import jax.numpy as jnp
from jax import lax
from jax.experimental import pallas as pl
from jax.experimental.pallas import tpu as pltpu
from jax.experimental.pallas import tpu_sc as plsc

N = 10000
E = 320000
NFEAT = 128
NHID = 128
NCLASS = 64

NC = 2
NS = 16
CHUNK = 128
CHUNKS_PER_TILE = 160
E_PAD = NS * CHUNK * CHUNKS_PER_TILE
HALF_CHUNKS = CHUNKS_PER_TILE // 2
NPAD = 10240
ROWS_PER_TILE = NPAD // NS
ZROWS = 64


def _sc_aggregate(cw, with_degree):
    mesh = plsc.VectorSubcoreMesh(core_axis_name="c", subcore_axis_name="s")
    out_type = [jax.ShapeDtypeStruct((NC, NPAD, cw), jnp.float32)]
    scratch = [
        pltpu.VMEM_SHARED((NPAD, cw), jnp.float32),
        pltpu.VMEM((CHUNKS_PER_TILE, CHUNK), jnp.int32),
        pltpu.VMEM((CHUNKS_PER_TILE, CHUNK), jnp.int32),
        pltpu.VMEM((CHUNK, cw), jnp.float32),
        pltpu.VMEM((CHUNK, cw), jnp.float32),
        pltpu.VMEM((ZROWS, cw), jnp.float32),
        pltpu.SemaphoreType.DMA,
        pltpu.SemaphoreType.DMA,
        pltpu.SemaphoreType.DMA,
    ]
    if with_degree:
        out_type.append(jax.ShapeDtypeStruct((NC, NPAD, 16), jnp.float32))
        scratch += [
            pltpu.VMEM_SHARED((NPAD, 16), jnp.float32),
            pltpu.VMEM((CHUNK, 16), jnp.float32),
            pltpu.VMEM((ZROWS, 16), jnp.float32),
            pltpu.SemaphoreType.DMA,
        ]

    @functools.partial(
        pl.kernel,
        out_type=tuple(out_type),
        mesh=mesh,
        scratch_types=tuple(scratch),
        compiler_params=pltpu.CompilerParams(use_tc_tiling_on_sc=False),
    )
    def k(table_hbm, src4_hbm, dst3_hbm, *refs):
        if with_degree:
            (out_hbm, deg_hbm, acc_sh, src_v, dst_v, r0, r1, zbuf,
             g0, g1, zsem, deg_sh, ones_v, zbufd, dsem) = refs
        else:
            (out_hbm, acc_sh, src_v, dst_v, r0, r1, zbuf,
             g0, g1, zsem) = refs
        rows = [r0, r1]
        gsem = [g0, g1]

        cid = lax.axis_index("c")
        sid = lax.axis_index("s")

        z = jnp.zeros((16,), jnp.float32)

        def fill_z(i, _):
            for j in range(cw // 16):
                zbuf[i, pl.ds(j * 16, 16)] = z
            if with_degree:
                zbufd[i, :] = z
            return 0

        lax.fori_loop(0, ZROWS, fill_z, 0)

        if with_degree:
            one = jnp.ones((16,), jnp.float32)

            def fill_ones(i, _):
                ones_v[i, :] = one
                return 0

            lax.fori_loop(0, CHUNK, fill_ones, 0)

        row0 = sid * ROWS_PER_TILE
        nz = ROWS_PER_TILE // ZROWS

        def zero_body(i, _):
            pltpu.async_copy(zbuf, acc_sh.at[pl.ds(row0 + i * ZROWS, ZROWS)],
                             zsem)
            if with_degree:
                pltpu.async_copy(
                    zbufd, deg_sh.at[pl.ds(row0 + i * ZROWS, ZROWS)], zsem)
            return 0

        lax.fori_loop(0, nz, zero_body, 0)

        pltpu.sync_copy(src4_hbm.at[cid, sid], src_v)
        pltpu.sync_copy(dst3_hbm.at[sid], dst_v)

        def zero_drain(i, _):
            pltpu.make_async_copy(
                zbuf, acc_sh.at[pl.ds(row0, ZROWS)], zsem).wait()
            if with_degree:
                pltpu.make_async_copy(
                    zbufd, deg_sh.at[pl.ds(row0, ZROWS)], zsem).wait()
            return 0

        lax.fori_loop(0, nz, zero_drain, 0)

        plsc.subcore_barrier()

        def fire_g(c, k):
            pltpu.async_copy(table_hbm.at[src_v.at[c]], rows[k], gsem[k])

        def wait_g(k):
            pltpu.make_async_copy(
                table_hbm.at[src_v.at[0]], rows[k], gsem[k]).wait()

        def scat(c, k):
            pltpu.sync_copy(rows[k], acc_sh.at[dst_v.at[c]], add=True)
            if with_degree:
                do = jnp.logical_or(
                    jnp.logical_and(cid == 0, c < HALF_CHUNKS),
                    jnp.logical_and(cid != 0, c >= HALF_CHUNKS))

                @pl.when(do)
                def _():
                    pltpu.async_copy(ones_v, deg_sh.at[dst_v.at[c]], dsem,
                                     add=True)

        fire_g(0, 0)
        NPAIR = CHUNKS_PER_TILE // 2

        def pair_body(g, _):
            c0 = 2 * g
            fire_g(c0 + 1, 1)
            wait_g(0)
            scat(c0, 0)

            @pl.when(g + 1 < NPAIR)
            def _():
                fire_g(c0 + 2, 0)

            wait_g(1)
            scat(c0 + 1, 1)
            return 0

        lax.fori_loop(0, NPAIR, pair_body, 0)

        if with_degree:
            def deg_drain(i, _):
                pltpu.make_async_copy(
                    ones_v, deg_sh.at[dst_v.at[0]], dsem).wait()
                return 0

            lax.fori_loop(0, HALF_CHUNKS, deg_drain, 0)

        plsc.subcore_barrier()

        pltpu.async_copy(
            acc_sh.at[pl.ds(row0, ROWS_PER_TILE)],
            out_hbm.at[cid, pl.ds(row0, ROWS_PER_TILE)],
            zsem)
        if with_degree:
            pltpu.async_copy(
                deg_sh.at[pl.ds(row0, ROWS_PER_TILE)],
                deg_hbm.at[cid, pl.ds(row0, ROWS_PER_TILE)],
                zsem)
            pltpu.make_async_copy(
                deg_sh.at[pl.ds(row0, ROWS_PER_TILE)],
                deg_hbm.at[cid, pl.ds(row0, ROWS_PER_TILE)],
                zsem).wait()
        pltpu.make_async_copy(
            acc_sh.at[pl.ds(row0, ROWS_PER_TILE)],
            out_hbm.at[cid, pl.ds(row0, ROWS_PER_TILE)],
            zsem).wait()

    return k


_sc_agg_l1 = _sc_aggregate(NFEAT // NC, with_degree=True)
_sc_agg_l2 = _sc_aggregate(NCLASS // NC, with_degree=False)

BR = 512
CW1 = NFEAT // NC
CW2 = NCLASS // NC


def _tc_layer1_body(agg_ref, deg_ref, x_ref, w1l_ref, b1_ref, w1r_ref,
                    w2l_ref, w2r_ref, hl_ref, hr_ref):
    agg = jnp.concatenate([agg_ref[0], agg_ref[1]], axis=1)
    deg = deg_ref[0, :, :1] + deg_ref[1, :, :1]
    inv = 1.0 / jnp.maximum(deg, 1.0)
    mean = agg * inv
    dn = (((1,), (1,)), ((), ()))
    h = (lax.dot_general(mean, w1l_ref[...], dn,
                         preferred_element_type=jnp.float32)
         + b1_ref[...]
         + lax.dot_general(x_ref[...], w1r_ref[...], dn,
                           preferred_element_type=jnp.float32))
    hl = lax.dot_general(h, w2l_ref[...], dn,
                         preferred_element_type=jnp.float32)
    hl_ref[0] = hl[:, :CW2]
    hl_ref[1] = hl[:, CW2:]
    hr_ref[...] = lax.dot_general(h, w2r_ref[...], dn,
                                  preferred_element_type=jnp.float32)


def _tc_layer2_body(agg_ref, deg_ref, hr_ref, b2_ref, out_ref):
    agg = jnp.concatenate([agg_ref[0], agg_ref[1]], axis=1)
    deg = deg_ref[0, :, :1] + deg_ref[1, :, :1]
    inv = 1.0 / jnp.maximum(deg, 1.0)
    z = agg * inv + b2_ref[...] + hr_ref[...]
    m = jnp.max(z, axis=1, keepdims=True)
    lse = m + jnp.log(jnp.sum(jnp.exp(z - m), axis=1, keepdims=True))
    out_ref[...] = z - lse


def _tc_layer1(agg1, deg, x_pad, W1l, b1, W1r, W2l, W2r):
    grid = (NPAD // BR,)
    return pl.pallas_call(
        _tc_layer1_body,
        grid=grid,
        in_specs=[
            pl.BlockSpec((NC, BR, CW1), lambda r: (0, r, 0)),
            pl.BlockSpec((NC, BR, 16), lambda r: (0, r, 0)),
            pl.BlockSpec((BR, NFEAT), lambda r: (r, 0)),
            pl.BlockSpec((NHID, NFEAT), lambda r: (0, 0)),
            pl.BlockSpec((1, NHID), lambda r: (0, 0)),
            pl.BlockSpec((NHID, NFEAT), lambda r: (0, 0)),
            pl.BlockSpec((NCLASS, NHID), lambda r: (0, 0)),
            pl.BlockSpec((NCLASS, NHID), lambda r: (0, 0)),
        ],
        out_specs=[
            pl.BlockSpec((NC, BR, CW2), lambda r: (0, r, 0)),
            pl.BlockSpec((BR, NCLASS), lambda r: (r, 0)),
        ],
        out_shape=[
            jax.ShapeDtypeStruct((NC, NPAD, CW2), jnp.float32),
            jax.ShapeDtypeStruct((NPAD, NCLASS), jnp.float32),
        ],
    )(agg1, deg, x_pad, W1l, b1, W1r, W2l, W2r)


def _tc_layer2(agg2, deg, hr, b2):
    grid = (NPAD // BR,)
    return pl.pallas_call(
        _tc_layer2_body,
        grid=grid,
        in_specs=[
            pl.BlockSpec((NC, BR, CW2), lambda r: (0, r, 0)),
            pl.BlockSpec((NC, BR, 16), lambda r: (0, r, 0)),
            pl.BlockSpec((BR, NCLASS), lambda r: (r, 0)),
            pl.BlockSpec((1, NCLASS), lambda r: (0, 0)),
        ],
        out_specs=pl.BlockSpec((BR, NCLASS), lambda r: (r, 0)),
        out_shape=jax.ShapeDtypeStruct((NPAD, NCLASS), jnp.float32),
    )(agg2, deg, hr, b2)


@jax.jit
def kernel(x, edge_index, W1l, b1, W1r, W2l, b2, W2r):
    src = edge_index[0]
    dst = edge_index[1]
    pad = E_PAD - E
    srcp = jnp.concatenate([src, jnp.arange(pad, dtype=jnp.int32) % N])
    dummy = N + jnp.arange(pad, dtype=jnp.int32) % (NPAD - N)
    dstp = jnp.concatenate([dst, dummy])
    src3 = srcp.reshape(NS, CHUNKS_PER_TILE, CHUNK)
    src4 = jnp.stack([src3, src3 + NPAD])
    dst3 = dstp.reshape(NS, CHUNKS_PER_TILE, CHUNK)

    x_pad = jnp.pad(x, ((0, NPAD - N), (0, 0)))
    xcat = jnp.concatenate([x_pad[:, :CW1], x_pad[:, CW1:]], axis=0)

    agg1, deg = _sc_agg_l1(xcat, src4, dst3)

    hl, hr = _tc_layer1(agg1, deg, x_pad, W1l, b1.reshape(1, NHID), W1r,
                        W2l, W2r)

    (agg2,) = _sc_agg_l2(hl.reshape(NC * NPAD, CW2), src4, dst3)

    out = _tc_layer2(agg2, deg, hr, b2.reshape(1, NCLASS))
    return out[:N]


# speedup vs baseline: 1.9164x; 1.0426x over previous
import functools

import jax
import jax.numpy as jnp
from jax import lax
from jax.experimental import pallas as pl
from jax.experimental.pallas import tpu as pltpu
from jax.experimental.pallas import tpu_sc as plsc

N = 10000
E = 320000
NFEAT = 128
NHID = 128
NCLASS = 64

NC = 2
NS = 16
CHUNK = 128
CHUNKS_PER_TILE = 160
E_PAD = NS * CHUNK * CHUNKS_PER_TILE
HALF_CHUNKS = CHUNKS_PER_TILE // 2
NPAD = 10240
ROWS_PER_TILE = NPAD // NS
ZROWS = 32


def _sc_aggregate(cw, with_degree):
    mesh = plsc.VectorSubcoreMesh(core_axis_name="c", subcore_axis_name="s")
    out_type = [jax.ShapeDtypeStruct((NC, NPAD, cw), jnp.float32)]
    scratch = [
        pltpu.VMEM_SHARED((NPAD, cw), jnp.float32),
        pltpu.VMEM((CHUNKS_PER_TILE, CHUNK), jnp.int32),
        pltpu.VMEM((CHUNKS_PER_TILE, CHUNK), jnp.int32),
        *[pltpu.VMEM((CHUNK, cw), jnp.float32) for _ in range(4)],
        pltpu.VMEM((ZROWS, cw), jnp.float32),
        *[pltpu.SemaphoreType.DMA for _ in range(4)],
        *[pltpu.SemaphoreType.DMA for _ in range(4)],
        pltpu.SemaphoreType.DMA,
    ]
    if with_degree:
        out_type.append(jax.ShapeDtypeStruct((NC, NPAD, 16), jnp.float32))
        scratch += [
            pltpu.VMEM_SHARED((NPAD, 16), jnp.float32),
            pltpu.VMEM((CHUNK, 16), jnp.float32),
            pltpu.VMEM((ZROWS, 16), jnp.float32),
            pltpu.SemaphoreType.DMA,
        ]

    @functools.partial(
        pl.kernel,
        out_type=tuple(out_type),
        mesh=mesh,
        scratch_types=tuple(scratch),
        compiler_params=pltpu.CompilerParams(use_tc_tiling_on_sc=False),
    )
    def k(table_hbm, src4_hbm, dst3_hbm, *refs):
        if with_degree:
            (out_hbm, deg_hbm, acc_sh, src_v, dst_v, r0, r1, r2, r3, zbuf,
             g0, g1, g2, g3, s0, s1, s2, s3, zsem,
             deg_sh, ones_v, zbufd, dsem) = refs
        else:
            (out_hbm, acc_sh, src_v, dst_v, r0, r1, r2, r3, zbuf,
             g0, g1, g2, g3, s0, s1, s2, s3, zsem) = refs
        rows = [r0, r1, r2, r3]
        gsem = [g0, g1, g2, g3]
        ssem = [s0, s1, s2, s3]

        cid = lax.axis_index("c")
        sid = lax.axis_index("s")

        z = jnp.zeros((16,), jnp.float32)

        def fill_z(i, _):
            for j in range(cw // 16):
                zbuf[i, pl.ds(j * 16, 16)] = z
            if with_degree:
                zbufd[i, :] = z
            return 0

        lax.fori_loop(0, ZROWS, fill_z, 0)

        if with_degree:
            one = jnp.ones((16,), jnp.float32)

            def fill_ones(i, _):
                ones_v[i, :] = one
                return 0

            lax.fori_loop(0, CHUNK, fill_ones, 0)

        row0 = sid * ROWS_PER_TILE
        nz = ROWS_PER_TILE // ZROWS

        def zero_body(i, _):
            pltpu.async_copy(zbuf, acc_sh.at[pl.ds(row0 + i * ZROWS, ZROWS)],
                             zsem)
            if with_degree:
                pltpu.async_copy(
                    zbufd, deg_sh.at[pl.ds(row0 + i * ZROWS, ZROWS)], zsem)
            return 0

        lax.fori_loop(0, nz, zero_body, 0)

        pltpu.sync_copy(src4_hbm.at[cid, sid], src_v)
        pltpu.sync_copy(dst3_hbm.at[sid], dst_v)

        def zero_drain(i, _):
            pltpu.make_async_copy(
                zbuf, acc_sh.at[pl.ds(row0, ZROWS)], zsem).wait()
            if with_degree:
                pltpu.make_async_copy(
                    zbufd, deg_sh.at[pl.ds(row0, ZROWS)], zsem).wait()
            return 0

        lax.fori_loop(0, nz, zero_drain, 0)

        plsc.subcore_barrier()

        def fire_g(c, k):
            pltpu.async_copy(table_hbm.at[src_v.at[c]], rows[k], gsem[k])

        def wait_g(k):
            pltpu.make_async_copy(
                table_hbm.at[src_v.at[0]], rows[k], gsem[k]).wait()

        def fire_s(c, k):
            pltpu.async_copy(rows[k], acc_sh.at[dst_v.at[c]], ssem[k],
                             add=True)
            if with_degree:
                do = jnp.logical_or(
                    jnp.logical_and(cid == 0, c < HALF_CHUNKS),
                    jnp.logical_and(cid != 0, c >= HALF_CHUNKS))

                @pl.when(do)
                def _():
                    pltpu.async_copy(ones_v, deg_sh.at[dst_v.at[c]], dsem,
                                     add=True)

        def wait_s(k):
            pltpu.make_async_copy(
                rows[k], acc_sh.at[dst_v.at[0]], ssem[k]).wait()

        fire_g(0, 0)
        fire_g(1, 1)
        NGRP = CHUNKS_PER_TILE // 4

        def grp_body(g, _):
            for k in range(4):
                c = 4 * g + k
                wait_g(k)
                fire_s(c, k)
                k2 = (k + 2) % 4
                if k < 2:
                    @pl.when(g > 0)
                    def _():
                        wait_s(k2)

                    fire_g(c + 2, k2)
                else:
                    wait_s(k2)

                    @pl.when(g < NGRP - 1)
                    def _():
                        fire_g(c + 2, k2)
            return 0

        lax.fori_loop(0, NGRP, grp_body, 0)

        wait_s((CHUNKS_PER_TILE - 2) % 4)
        wait_s((CHUNKS_PER_TILE - 1) % 4)

        if with_degree:
            def deg_drain(i, _):
                pltpu.make_async_copy(
                    ones_v, deg_sh.at[dst_v.at[0]], dsem).wait()
                return 0

            lax.fori_loop(0, HALF_CHUNKS, deg_drain, 0)

        plsc.subcore_barrier()

        pltpu.async_copy(
            acc_sh.at[pl.ds(row0, ROWS_PER_TILE)],
            out_hbm.at[cid, pl.ds(row0, ROWS_PER_TILE)],
            zsem)
        if with_degree:
            pltpu.async_copy(
                deg_sh.at[pl.ds(row0, ROWS_PER_TILE)],
                deg_hbm.at[cid, pl.ds(row0, ROWS_PER_TILE)],
                zsem)
            pltpu.make_async_copy(
                deg_sh.at[pl.ds(row0, ROWS_PER_TILE)],
                deg_hbm.at[cid, pl.ds(row0, ROWS_PER_TILE)],
                zsem).wait()
        pltpu.make_async_copy(
            acc_sh.at[pl.ds(row0, ROWS_PER_TILE)],
            out_hbm.at[cid, pl.ds(row0, ROWS_PER_TILE)],
            zsem).wait()

    return k


_sc_agg_l1 = _sc_aggregate(NFEAT // NC, with_degree=True)
_sc_agg_l2 = _sc_aggregate(NCLASS // NC, with_degree=False)

BR = 512
CW1 = NFEAT // NC
CW2 = NCLASS // NC


def _tc_layer1_body(agg_ref, deg_ref, x_ref, w1l_ref, b1_ref, w1r_ref,
                    w2l_ref, w2r_ref, hl_ref, hr_ref):
    agg = jnp.concatenate([agg_ref[0], agg_ref[1]], axis=1)
    deg = deg_ref[0, :, :1] + deg_ref[1, :, :1]
    inv = 1.0 / jnp.maximum(deg, 1.0)
    mean = agg * inv
    dn = (((1,), (1,)), ((), ()))
    h = (lax.dot_general(mean, w1l_ref[...], dn,
                         preferred_element_type=jnp.float32)
         + b1_ref[...]
         + lax.dot_general(x_ref[...], w1r_ref[...], dn,
                           preferred_element_type=jnp.float32))
    hl = lax.dot_general(h, w2l_ref[...], dn,
                         preferred_element_type=jnp.float32)
    hl_ref[0] = hl[:, :CW2]
    hl_ref[1] = hl[:, CW2:]
    hr_ref[...] = lax.dot_general(h, w2r_ref[...], dn,
                                  preferred_element_type=jnp.float32)


def _tc_layer2_body(agg_ref, deg_ref, hr_ref, b2_ref, out_ref):
    agg = jnp.concatenate([agg_ref[0], agg_ref[1]], axis=1)
    deg = deg_ref[0, :, :1] + deg_ref[1, :, :1]
    inv = 1.0 / jnp.maximum(deg, 1.0)
    z = agg * inv + b2_ref[...] + hr_ref[...]
    m = jnp.max(z, axis=1, keepdims=True)
    lse = m + jnp.log(jnp.sum(jnp.exp(z - m), axis=1, keepdims=True))
    out_ref[...] = z - lse


def _tc_layer1(agg1, deg, x_pad, W1l, b1, W1r, W2l, W2r):
    grid = (NPAD // BR,)
    return pl.pallas_call(
        _tc_layer1_body,
        grid=grid,
        in_specs=[
            pl.BlockSpec((NC, BR, CW1), lambda r: (0, r, 0)),
            pl.BlockSpec((NC, BR, 16), lambda r: (0, r, 0)),
            pl.BlockSpec((BR, NFEAT), lambda r: (r, 0)),
            pl.BlockSpec((NHID, NFEAT), lambda r: (0, 0)),
            pl.BlockSpec((1, NHID), lambda r: (0, 0)),
            pl.BlockSpec((NHID, NFEAT), lambda r: (0, 0)),
            pl.BlockSpec((NCLASS, NHID), lambda r: (0, 0)),
            pl.BlockSpec((NCLASS, NHID), lambda r: (0, 0)),
        ],
        out_specs=[
            pl.BlockSpec((NC, BR, CW2), lambda r: (0, r, 0)),
            pl.BlockSpec((BR, NCLASS), lambda r: (r, 0)),
        ],
        out_shape=[
            jax.ShapeDtypeStruct((NC, NPAD, CW2), jnp.float32),
            jax.ShapeDtypeStruct((NPAD, NCLASS), jnp.float32),
        ],
    )(agg1, deg, x_pad, W1l, b1, W1r, W2l, W2r)


def _tc_layer2(agg2, deg, hr, b2):
    grid = (NPAD // BR,)
    return pl.pallas_call(
        _tc_layer2_body,
        grid=grid,
        in_specs=[
            pl.BlockSpec((NC, BR, CW2), lambda r: (0, r, 0)),
            pl.BlockSpec((NC, BR, 16), lambda r: (0, r, 0)),
            pl.BlockSpec((BR, NCLASS), lambda r: (r, 0)),
            pl.BlockSpec((1, NCLASS), lambda r: (0, 0)),
        ],
        out_specs=pl.BlockSpec((BR, NCLASS), lambda r: (r, 0)),
        out_shape=jax.ShapeDtypeStruct((NPAD, NCLASS), jnp.float32),
    )(agg2, deg, hr, b2)


@jax.jit
def kernel(x, edge_index, W1l, b1, W1r, W2l, b2, W2r):
    src = edge_index[0]
    dst = edge_index[1]
    pad = E_PAD - E
    srcp = jnp.concatenate([src, jnp.arange(pad, dtype=jnp.int32) % N])
    dummy = N + jnp.arange(pad, dtype=jnp.int32) % (NPAD - N)
    dstp = jnp.concatenate([dst, dummy])
    src3 = srcp.reshape(NS, CHUNKS_PER_TILE, CHUNK)
    src4 = jnp.stack([src3, src3 + NPAD])
    dst3 = dstp.reshape(NS, CHUNKS_PER_TILE, CHUNK)

    x_pad = jnp.pad(x, ((0, NPAD - N), (0, 0)))
    xcat = jnp.concatenate([x_pad[:, :CW1], x_pad[:, CW1:]], axis=0)

    agg1, deg = _sc_agg_l1(xcat, src4, dst3)

    hl, hr = _tc_layer1(agg1, deg, x_pad, W1l, b1.reshape(1, NHID), W1r,
                        W2l, W2r)

    (agg2,) = _sc_agg_l2(hl.reshape(NC * NPAD, CW2), src4, dst3)

    out = _tc_layer2(agg2, deg, hr, b2.reshape(1, NCLASS))
    return out[:N]


# speedup vs baseline: 2.1226x; 1.1076x over previous
import functools

import jax
import jax.numpy as jnp
from jax import lax
from jax.experimental import pallas as pl
from jax.experimental.pallas import tpu as pltpu
from jax.experimental.pallas import tpu_sc as plsc

N = 10000
E = 320000
NFEAT = 128
NHID = 128
NCLASS = 64

NC = 2
NS = 16
CHUNK = 128
CHUNKS_PER_TILE = 160
E_PAD = NS * CHUNK * CHUNKS_PER_TILE
HALF_CHUNKS = CHUNKS_PER_TILE // 2
NPAD = 10240
ROWS_PER_TILE = NPAD // NS
ZROWS = 32


def _sc_aggregate(cw, mul, coff, with_degree):
    mesh = plsc.VectorSubcoreMesh(core_axis_name="c", subcore_axis_name="s")
    out_type = [jax.ShapeDtypeStruct((NC, NPAD, cw), jnp.float32)]
    scratch = [
        pltpu.VMEM_SHARED((NPAD, cw), jnp.float32),
        pltpu.VMEM((CHUNKS_PER_TILE, CHUNK), jnp.int32),
        pltpu.VMEM((CHUNKS_PER_TILE, CHUNK), jnp.int32),
        *[pltpu.VMEM((CHUNK, cw), jnp.float32) for _ in range(4)],
        pltpu.VMEM((ZROWS, cw), jnp.float32),
        *[pltpu.SemaphoreType.DMA for _ in range(4)],
        *[pltpu.SemaphoreType.DMA for _ in range(4)],
        pltpu.SemaphoreType.DMA,
    ]
    if with_degree:
        out_type.append(jax.ShapeDtypeStruct((NC, NPAD, 16), jnp.float32))
        scratch += [
            pltpu.VMEM_SHARED((NPAD, 16), jnp.float32),
            pltpu.VMEM((CHUNK, 16), jnp.float32),
            pltpu.VMEM((ZROWS, 16), jnp.float32),
            pltpu.SemaphoreType.DMA,
        ]

    @functools.partial(
        pl.kernel,
        out_type=tuple(out_type),
        mesh=mesh,
        scratch_types=tuple(scratch),
        compiler_params=pltpu.CompilerParams(use_tc_tiling_on_sc=False),
    )
    def k(table_hbm, src3_hbm, dst3_hbm, *refs):
        if with_degree:
            (out_hbm, deg_hbm, acc_sh, src_v, dst_v, r0, r1, r2, r3, zbuf,
             g0, g1, g2, g3, s0, s1, s2, s3, zsem,
             deg_sh, ones_v, zbufd, dsem) = refs
        else:
            (out_hbm, acc_sh, src_v, dst_v, r0, r1, r2, r3, zbuf,
             g0, g1, g2, g3, s0, s1, s2, s3, zsem) = refs
        rows = [r0, r1, r2, r3]
        gsem = [g0, g1, g2, g3]
        ssem = [s0, s1, s2, s3]

        cid = lax.axis_index("c")
        sid = lax.axis_index("s")

        z = jnp.zeros((16,), jnp.float32)

        def fill_z(i, _):
            for j in range(cw // 16):
                zbuf[i, pl.ds(j * 16, 16)] = z
            if with_degree:
                zbufd[i, :] = z
            return 0

        lax.fori_loop(0, ZROWS, fill_z, 0)

        if with_degree:
            one = jnp.ones((16,), jnp.float32)

            def fill_ones(i, _):
                ones_v[i, :] = one
                return 0

            lax.fori_loop(0, CHUNK, fill_ones, 0)

        row0 = sid * ROWS_PER_TILE
        nz = ROWS_PER_TILE // ZROWS

        def zero_body(i, _):
            pltpu.async_copy(zbuf, acc_sh.at[pl.ds(row0 + i * ZROWS, ZROWS)],
                             zsem)
            if with_degree:
                pltpu.async_copy(
                    zbufd, deg_sh.at[pl.ds(row0 + i * ZROWS, ZROWS)], zsem)
            return 0

        lax.fori_loop(0, nz, zero_body, 0)

        pltpu.sync_copy(src3_hbm.at[sid], src_v)
        pltpu.sync_copy(dst3_hbm.at[sid], dst_v)

        def zero_drain(i, _):
            pltpu.make_async_copy(
                zbuf, acc_sh.at[pl.ds(row0, ZROWS)], zsem).wait()
            if with_degree:
                pltpu.make_async_copy(
                    zbufd, deg_sh.at[pl.ds(row0, ZROWS)], zsem).wait()
            return 0

        lax.fori_loop(0, nz, zero_drain, 0)

        cadd = cid * coff

        def xform(i, _):
            for j in range(CHUNK // 16):
                sl = src_v[i, pl.ds(j * 16, 16)]
                src_v[i, pl.ds(j * 16, 16)] = sl * mul + cadd
            return 0

        lax.fori_loop(0, CHUNKS_PER_TILE, xform, 0)

        plsc.subcore_barrier()

        def fire_g(c, k):
            pltpu.async_copy(table_hbm.at[src_v.at[c]], rows[k], gsem[k])

        def wait_g(k):
            pltpu.make_async_copy(
                table_hbm.at[src_v.at[0]], rows[k], gsem[k]).wait()

        def fire_s(c, k):
            pltpu.async_copy(rows[k], acc_sh.at[dst_v.at[c]], ssem[k],
                             add=True)
            if with_degree:
                do = jnp.logical_or(
                    jnp.logical_and(cid == 0, c < HALF_CHUNKS),
                    jnp.logical_and(cid != 0, c >= HALF_CHUNKS))

                @pl.when(do)
                def _():
                    pltpu.async_copy(ones_v, deg_sh.at[dst_v.at[c]], dsem,
                                     add=True)

        def wait_s(k):
            pltpu.make_async_copy(
                rows[k], acc_sh.at[dst_v.at[0]], ssem[k]).wait()

        fire_g(0, 0)
        fire_g(1, 1)
        NGRP = CHUNKS_PER_TILE // 4

        def grp_body(g, _):
            for k in range(4):
                c = 4 * g + k
                wait_g(k)
                fire_s(c, k)
                k2 = (k + 2) % 4
                if k < 2:
                    @pl.when(g > 0)
                    def _():
                        wait_s(k2)

                    fire_g(c + 2, k2)
                else:
                    wait_s(k2)

                    @pl.when(g < NGRP - 1)
                    def _():
                        fire_g(c + 2, k2)
            return 0

        lax.fori_loop(0, NGRP, grp_body, 0)

        wait_s((CHUNKS_PER_TILE - 2) % 4)
        wait_s((CHUNKS_PER_TILE - 1) % 4)

        if with_degree:
            def deg_drain(i, _):
                pltpu.make_async_copy(
                    ones_v, deg_sh.at[dst_v.at[0]], dsem).wait()
                return 0

            lax.fori_loop(0, HALF_CHUNKS, deg_drain, 0)

        plsc.subcore_barrier()

        pltpu.async_copy(
            acc_sh.at[pl.ds(row0, ROWS_PER_TILE)],
            out_hbm.at[cid, pl.ds(row0, ROWS_PER_TILE)],
            zsem)
        if with_degree:
            pltpu.async_copy(
                deg_sh.at[pl.ds(row0, ROWS_PER_TILE)],
                deg_hbm.at[cid, pl.ds(row0, ROWS_PER_TILE)],
                zsem)
            pltpu.make_async_copy(
                deg_sh.at[pl.ds(row0, ROWS_PER_TILE)],
                deg_hbm.at[cid, pl.ds(row0, ROWS_PER_TILE)],
                zsem).wait()
        pltpu.make_async_copy(
            acc_sh.at[pl.ds(row0, ROWS_PER_TILE)],
            out_hbm.at[cid, pl.ds(row0, ROWS_PER_TILE)],
            zsem).wait()

    return k


_sc_agg_l1 = _sc_aggregate(NFEAT // NC, 2, 1, with_degree=True)
_sc_agg_l2 = _sc_aggregate(NCLASS // NC, 1, N, with_degree=False)

BR = 1000
NBLK = N // BR
CW1 = NFEAT // NC
CW2 = NCLASS // NC

_DN = (((1,), (1,)), ((), ()))


def _tc_self_body(x_ref, w1r_ref, b1_ref, xr_ref):
    xr_ref[...] = lax.dot_general(
        x_ref[...], w1r_ref[...], _DN,
        preferred_element_type=jnp.float32) + b1_ref[...]


def _tc_main_body(agg_ref, deg_ref, xr_ref, w1l_ref, w2l_ref, w2r_ref,
                  hl_ref, hr_ref):
    agg = jnp.concatenate([agg_ref[0], agg_ref[1]], axis=1)
    deg = deg_ref[0, :, :1] + deg_ref[1, :, :1]
    inv = 1.0 / jnp.maximum(deg, 1.0)
    mean = agg * inv
    h = lax.dot_general(mean, w1l_ref[...], _DN,
                        preferred_element_type=jnp.float32) + xr_ref[...]
    hl = lax.dot_general(h, w2l_ref[...], _DN,
                         preferred_element_type=jnp.float32)
    hl_ref[0] = hl[:, :CW2]
    hl_ref[1] = hl[:, CW2:]
    hr_ref[...] = lax.dot_general(h, w2r_ref[...], _DN,
                                  preferred_element_type=jnp.float32)


def _tc_final_body(agg_ref, deg_ref, hr_ref, b2_ref, out_ref):
    agg = jnp.concatenate([agg_ref[0], agg_ref[1]], axis=1)
    deg = deg_ref[0, :, :1] + deg_ref[1, :, :1]
    inv = 1.0 / jnp.maximum(deg, 1.0)
    z = agg * inv + b2_ref[...] + hr_ref[...]
    m = jnp.max(z, axis=1, keepdims=True)
    lse = m + jnp.log(jnp.sum(jnp.exp(z - m), axis=1, keepdims=True))
    out_ref[...] = z - lse


def _tc_self(x, W1r, b1):
    return pl.pallas_call(
        _tc_self_body,
        grid=(NBLK,),
        in_specs=[
            pl.BlockSpec((BR, NFEAT), lambda r: (r, 0)),
            pl.BlockSpec((NHID, NFEAT), lambda r: (0, 0)),
            pl.BlockSpec((1, NHID), lambda r: (0, 0)),
        ],
        out_specs=pl.BlockSpec((BR, NHID), lambda r: (r, 0)),
        out_shape=jax.ShapeDtypeStruct((N, NHID), jnp.float32),
    )(x, W1r, b1)


def _tc_main(agg1, deg, xr, W1l, W2l, W2r):
    return pl.pallas_call(
        _tc_main_body,
        grid=(NBLK,),
        in_specs=[
            pl.BlockSpec((NC, BR, CW1), lambda r: (0, r, 0)),
            pl.BlockSpec((NC, BR, 16), lambda r: (0, r, 0)),
            pl.BlockSpec((BR, NHID), lambda r: (r, 0)),
            pl.BlockSpec((NHID, NFEAT), lambda r: (0, 0)),
            pl.BlockSpec((NCLASS, NHID), lambda r: (0, 0)),
            pl.BlockSpec((NCLASS, NHID), lambda r: (0, 0)),
        ],
        out_specs=[
            pl.BlockSpec((NC, BR, CW2), lambda r: (0, r, 0)),
            pl.BlockSpec((BR, NCLASS), lambda r: (r, 0)),
        ],
        out_shape=[
            jax.ShapeDtypeStruct((NC, N, CW2), jnp.float32),
            jax.ShapeDtypeStruct((N, NCLASS), jnp.float32),
        ],
    )(agg1, deg, xr, W1l, W2l, W2r)


def _tc_final(agg2, deg, hr, b2):
    return pl.pallas_call(
        _tc_final_body,
        grid=(NBLK,),
        in_specs=[
            pl.BlockSpec((NC, BR, CW2), lambda r: (0, r, 0)),
            pl.BlockSpec((NC, BR, 16), lambda r: (0, r, 0)),
            pl.BlockSpec((BR, NCLASS), lambda r: (r, 0)),
            pl.BlockSpec((1, NCLASS), lambda r: (0, 0)),
        ],
        out_specs=pl.BlockSpec((BR, NCLASS), lambda r: (r, 0)),
        out_shape=jax.ShapeDtypeStruct((N, NCLASS), jnp.float32),
    )(agg2, deg, hr, b2)


@jax.jit
def kernel(x, edge_index, W1l, b1, W1r, W2l, b2, W2r):
    src = edge_index[0]
    dst = edge_index[1]
    pad = E_PAD - E
    srcp = jnp.concatenate([src, jnp.arange(pad, dtype=jnp.int32) % N])
    dummy = N + jnp.arange(pad, dtype=jnp.int32) % (NPAD - N)
    dstp = jnp.concatenate([dst, dummy])
    src3 = srcp.reshape(NS, CHUNKS_PER_TILE, CHUNK)
    dst3 = dstp.reshape(NS, CHUNKS_PER_TILE, CHUNK)

    agg1, deg = _sc_agg_l1(x.reshape(2 * N, CW1), src3, dst3)

    xr = _tc_self(x, W1r, b1.reshape(1, NHID))
    hl, hr = _tc_main(agg1, deg, xr, W1l, W2l, W2r)

    (agg2,) = _sc_agg_l2(hl.reshape(NC * N, CW2), src3, dst3)

    out = _tc_final(agg2, deg, hr, b2.reshape(1, NCLASS))
    return out
